# baseline (speedup 1.0000x reference)
import functools

import jax
import jax.numpy as jnp
from jax import lax
from jax.experimental import pallas as pl
from jax.experimental.pallas import tpu as pltpu
from jax.experimental.pallas import tpu_sc as plsc

VOCAB = 100000
DIM = 32
BATCH = 1024

_NC = 2
_NS = 16
_NW = _NC * _NS
_B_PER_W = BATCH // _NW


def _sc_gather(idx, table):
  @functools.partial(
      pl.kernel,
      mesh=plsc.VectorSubcoreMesh(core_axis_name="c", subcore_axis_name="s"),
      out_type=jax.ShapeDtypeStruct((BATCH, DIM), jnp.float32),
      scratch_types=[
          pltpu.VMEM((_B_PER_W,), jnp.int32),
          pltpu.VMEM((_B_PER_W, DIM), jnp.float32),
          pltpu.SemaphoreType.DMA,
      ],
      compiler_params=pltpu.CompilerParams(use_tc_tiling_on_sc=False),
  )
  def gather_kernel(idx_hbm, table_hbm, out_hbm, idx_v, rows_v, sem):
    wid = lax.axis_index("s") * _NC + lax.axis_index("c")
    base = wid * _B_PER_W
    pltpu.sync_copy(idx_hbm.at[pl.ds(base, _B_PER_W)], idx_v)
    pltpu.async_copy(table_hbm.at[idx_v], rows_v, sem).wait()
    pltpu.sync_copy(rows_v, out_hbm.at[pl.ds(base, _B_PER_W)])

  return gather_kernel(idx, table)


_VT = 2048
_NV = -(-VOCAB // _VT)


def _mm_body(x_ref, w_ref, b_ref, o_ref):
  o_ref[...] = (
      lax.dot_general(
          x_ref[...], w_ref[...],
          (((1,), (1,)), ((), ())),
          preferred_element_type=jnp.float32,
      )
      + b_ref[...]
  )


def _tc_matmul(x, w, bias_row):
  return pl.pallas_call(
      _mm_body,
      grid=(_NV,),
      in_specs=[
          pl.BlockSpec((BATCH, DIM), lambda i: (0, 0)),
          pl.BlockSpec((_VT, DIM), lambda i: (i, 0)),
          pl.BlockSpec((1, _VT), lambda i: (0, i)),
      ],
      out_specs=pl.BlockSpec((BATCH, _VT), lambda i: (0, i)),
      out_shape=jax.ShapeDtypeStruct((BATCH, VOCAB), jnp.float32),
  )(x, w, bias_row)


def kernel(input, emb_table, W, b):
  idx = input.reshape(BATCH).astype(jnp.int32)
  x = _sc_gather(idx, emb_table)
  return _tc_matmul(x, W, b.reshape(1, VOCAB))


# speedup vs baseline: 1.0790x; 1.0790x over previous
import functools

import jax
import jax.numpy as jnp
from jax import lax
from jax.experimental import pallas as pl
from jax.experimental.pallas import tpu as pltpu
from jax.experimental.pallas import tpu_sc as plsc

VOCAB = 100000
DIM = 32
BATCH = 1024

_NC = 2
_NS = 16
_NW = _NC * _NS
_B_PER_W = BATCH // _NW


def _sc_gather(idx, table):
  @functools.partial(
      pl.kernel,
      mesh=plsc.VectorSubcoreMesh(core_axis_name="c", subcore_axis_name="s"),
      out_type=jax.ShapeDtypeStruct((BATCH, DIM), jnp.float32),
      scratch_types=[
          pltpu.VMEM((_B_PER_W,), jnp.int32),
          pltpu.VMEM((_B_PER_W, DIM), jnp.float32),
          pltpu.SemaphoreType.DMA,
      ],
      compiler_params=pltpu.CompilerParams(use_tc_tiling_on_sc=False),
  )
  def gather_kernel(idx_hbm, table_hbm, out_hbm, idx_v, rows_v, sem):
    wid = lax.axis_index("s") * _NC + lax.axis_index("c")
    base = wid * _B_PER_W
    pltpu.sync_copy(idx_hbm.at[pl.ds(base, _B_PER_W)], idx_v)
    pltpu.async_copy(table_hbm.at[idx_v], rows_v, sem).wait()
    pltpu.sync_copy(rows_v, out_hbm.at[pl.ds(base, _B_PER_W)])

  return gather_kernel(idx, table)


_BM = 32
_NB = BATCH // _BM


def _mm_body(x_ref, wt_ref, b_ref, o_ref):
  o_ref[...] = (
      lax.dot_general(
          x_ref[...], wt_ref[...],
          (((1,), (0,)), ((), ())),
          preferred_element_type=jnp.float32,
      )
      + b_ref[...]
  )


def _tc_matmul_rows(x, wt, bias_row):
  return pl.pallas_call(
      _mm_body,
      grid=(_NB,),
      in_specs=[
          pl.BlockSpec((_BM, DIM), lambda i: (i, 0)),
          pl.BlockSpec((DIM, VOCAB), lambda i: (0, 0)),
          pl.BlockSpec((1, VOCAB), lambda i: (0, 0)),
      ],
      out_specs=pl.BlockSpec((_BM, VOCAB), lambda i: (i, 0)),
      out_shape=jax.ShapeDtypeStruct((BATCH, VOCAB), jnp.float32),
  )(x, wt, bias_row)


def kernel(input, emb_table, W, b):
  idx = input.reshape(BATCH).astype(jnp.int32)
  x = _sc_gather(idx, emb_table)
  return _tc_matmul_rows(x, W.T, b.reshape(1, VOCAB))


# speedup vs baseline: 1.9709x; 1.8266x over previous
import functools

import jax
import jax.numpy as jnp
from jax import lax
from jax.experimental import pallas as pl
from jax.experimental.pallas import tpu as pltpu
from jax.experimental.pallas import tpu_sc as plsc

VOCAB = 100000
DIM = 32
BATCH = 1024

_NC = 2
_NS = 16
_NW = _NC * _NS
_B_PER_W = BATCH // _NW


def _sc_gather(idx, table):
  @functools.partial(
      pl.kernel,
      mesh=plsc.VectorSubcoreMesh(core_axis_name="c", subcore_axis_name="s"),
      out_type=jax.ShapeDtypeStruct((BATCH, DIM), jnp.float32),
      scratch_types=[
          pltpu.VMEM((_B_PER_W,), jnp.int32),
          pltpu.VMEM((_B_PER_W, DIM), jnp.float32),
          pltpu.SemaphoreType.DMA,
      ],
      compiler_params=pltpu.CompilerParams(use_tc_tiling_on_sc=False),
  )
  def gather_kernel(idx_hbm, table_hbm, out_hbm, idx_v, rows_v, sem):
    wid = lax.axis_index("s") * _NC + lax.axis_index("c")
    base = wid * _B_PER_W
    pltpu.sync_copy(idx_hbm.at[pl.ds(base, _B_PER_W)], idx_v)
    pltpu.async_copy(table_hbm.at[idx_v], rows_v, sem).wait()
    pltpu.sync_copy(rows_v, out_hbm.at[pl.ds(base, _B_PER_W)])

  return gather_kernel(idx, table)


_VT = 2000
_NV = VOCAB // _VT


def _mm_body(w_ref, x_ref, b_ref, o_ref):
  o_ref[...] = (
      lax.dot_general(
          w_ref[...], x_ref[...],
          (((1,), (1,)), ((), ())),
          preferred_element_type=jnp.float32,
      )
      + b_ref[...]
  )


def _tc_matmul_t(x, w, bias_col):
  return pl.pallas_call(
      _mm_body,
      grid=(_NV,),
      in_specs=[
          pl.BlockSpec((_VT, DIM), lambda i: (i, 0)),
          pl.BlockSpec((BATCH, DIM), lambda i: (0, 0)),
          pl.BlockSpec((_VT, 1), lambda i: (i, 0)),
      ],
      out_specs=pl.BlockSpec((_VT, BATCH), lambda i: (i, 0)),
      out_shape=jax.ShapeDtypeStruct((VOCAB, BATCH), jnp.float32),
      compiler_params=pltpu.CompilerParams(
          dimension_semantics=("parallel",)
      ),
  )(w, x, bias_col)


def kernel(input, emb_table, W, b):
  idx = input.reshape(BATCH).astype(jnp.int32)
  x = _sc_gather(idx, emb_table)
  out_t = _tc_matmul_t(x, W, b.reshape(VOCAB, 1))
  return out_t.T
